# Initial kernel scaffold; baseline (speedup 1.0000x reference)
import jax
import jax.numpy as jnp
from jax.experimental import pallas as pl


def kernel(x, edge_index, batch, W0, b0, W1, b1, W2, b2, bn_gamma, bn_beta, Wg, att_src, att_dst, bg, Wc1, bc1, Wc2, bc2, Wc3, bc3):
    raise NotImplementedError("write your pallas kernel here")


# speedup vs baseline: 16.0093x; 16.0093x over previous
import functools

import jax
import jax.numpy as jnp
from jax import lax
from jax.experimental import pallas as pl
from jax.experimental.pallas import tpu as pltpu
from jax.experimental.pallas import tpu_sc as plsc

N = 10000
E = 320000
D = 128
H = 128
HEADS = 4
G = 512

NC = 2
NS = 16
NW = NC * NS
EB = 128
NBLK = 81
EPW = EB * NBLK
E2P = NW * EPW
NPAD = 10112
RT = NPAD // NS
RING = 8
PP = 10240
PRT = PP // NW
GPAD = 640
GRT = GPAD // NS
PW = 256

F32 = jnp.float32
I32 = jnp.int32


def _mesh():
    return plsc.VectorSubcoreMesh(
        core_axis_name="c", subcore_axis_name="s",
        num_cores=NC, num_subcores=NS)


@functools.partial(
    pl.kernel,
    out_type=jax.ShapeDtypeStruct((NC, NPAD, 128), F32),
    mesh=_mesh(),
    scratch_types=[
        pltpu.VMEM((EB, 128), F32),
        pltpu.VMEM((NBLK, EB), I32),
        pltpu.VMEM_SHARED((NPAD, 128), F32),
        pltpu.SemaphoreType.DMA,
    ],
)
def _deg_kernel(dst_hbm, ones_hbm, zeros_hbm, out_hbm, onesv, didx, acc, ssem):
    c = lax.axis_index("c")
    s = lax.axis_index("s")
    w = s * NC + c
    pltpu.sync_copy(ones_hbm, onesv)
    pltpu.sync_copy(dst_hbm.at[w], didx)
    pltpu.sync_copy(zeros_hbm, acc.at[pl.ds(s * RT, RT)])
    plsc.subcore_barrier()

    def blk(b, carry):
        pltpu.async_copy(onesv, acc.at[didx.at[b]], ssem, add=True)
        return carry

    lax.fori_loop(0, NBLK, blk, 0)

    def drain(b, carry):
        pltpu.make_async_copy(onesv, acc.at[didx.at[b]], ssem).wait()
        return carry

    lax.fori_loop(0, NBLK, drain, 0)
    plsc.subcore_barrier()
    pltpu.sync_copy(acc.at[pl.ds(s * RT, RT)], out_hbm.at[c, pl.ds(s * RT, RT)])


@functools.partial(
    pl.kernel,
    out_type=jax.ShapeDtypeStruct((NC, NPAD, 128), F32),
    mesh=_mesh(),
    scratch_types=[
        pltpu.VMEM((RING, EB), I32),
        pltpu.VMEM((NBLK, EB), I32),
        pltpu.VMEM((2, EB, 128), F32),
        pltpu.VMEM_SHARED((NPAD, 128), F32),
        pltpu.SemaphoreType.DMA,
        pltpu.SemaphoreType.DMA,
        pltpu.SemaphoreType.DMA,
    ],
)
def _gcn_kernel(xt_hbm, src_hbm, dst_hbm, zeros_hbm, out_hbm,
                sring, didx, rows, acc, isem, gsem, ssem):
    c = lax.axis_index("c")
    s = lax.axis_index("s")
    w = s * NC + c
    pltpu.sync_copy(dst_hbm.at[w], didx)
    pltpu.sync_copy(zeros_hbm, acc.at[pl.ds(s * RT, RT)])
    for k in range(RING):
        pltpu.async_copy(src_hbm.at[w, k], sring.at[k], isem)
    plsc.subcore_barrier()

    pltpu.make_async_copy(src_hbm.at[w, 0], sring.at[0], isem).wait()
    pltpu.async_copy(xt_hbm.at[sring.at[0]], rows.at[0], gsem)

    def blk(b, carry):
        slot = lax.rem(b, 2)
        ib = lax.rem(b, RING)
        ibn = lax.rem(b + 1, RING)
        pltpu.make_async_copy(
            xt_hbm.at[sring.at[ib]], rows.at[slot], gsem).wait()

        @pl.when(b >= 1)
        def _():
            pltpu.make_async_copy(
                rows.at[1 - slot], acc.at[didx.at[b - 1]], ssem).wait()

        @pl.when(b + 1 < NBLK)
        def _():
            pltpu.make_async_copy(
                src_hbm.at[w, b + 1], sring.at[ibn], isem).wait()
            pltpu.async_copy(xt_hbm.at[sring.at[ibn]], rows.at[1 - slot], gsem)

        pltpu.async_copy(rows.at[slot], acc.at[didx.at[b]], ssem, add=True)

        @pl.when(b + RING < NBLK)
        def _():
            pltpu.async_copy(src_hbm.at[w, b + RING], sring.at[ib], isem)

        return carry

    lax.fori_loop(0, NBLK, blk, 0)
    pltpu.make_async_copy(rows.at[lax.rem(NBLK - 1, 2)],
                          acc.at[didx.at[NBLK - 1]], ssem).wait()
    plsc.subcore_barrier()
    pltpu.sync_copy(acc.at[pl.ds(s * RT, RT)], out_hbm.at[c, pl.ds(s * RT, RT)])


@functools.partial(
    pl.kernel,
    out_type=jax.ShapeDtypeStruct((NW, NBLK, EB, 16), F32),
    mesh=_mesh(),
    scratch_types=[
        pltpu.VMEM((16,), F32),
        pltpu.VMEM((NBLK, EB), I32),
        pltpu.VMEM((NBLK, EB), I32),
        pltpu.VMEM((2, EB, 128), F32),
        pltpu.VMEM((2, EB, 128), F32),
        pltpu.VMEM((2, EB, 16), F32),
        pltpu.SemaphoreType.DMA,
        pltpu.SemaphoreType.DMA,
        pltpu.SemaphoreType.DMA,
    ],
)
def _ex_kernel(aas_hbm, aad_hbm, c_hbm, src_hbm, dst_hbm, ex_hbm,
               cv, sidx, didx, bufa, bufd, exb, asem, dsem, osem):
    c = lax.axis_index("c")
    s = lax.axis_index("s")
    w = s * NC + c
    pltpu.sync_copy(c_hbm, cv)
    pltpu.sync_copy(src_hbm.at[w], sidx)
    pltpu.sync_copy(dst_hbm.at[w], didx)
    cc = cv[...]
    pltpu.async_copy(aas_hbm.at[sidx.at[0]], bufa.at[0], asem)
    pltpu.async_copy(aad_hbm.at[didx.at[0]], bufd.at[0], dsem)

    def blk(b, carry):
        slot = lax.rem(b, 2)
        pltpu.make_async_copy(
            aas_hbm.at[sidx.at[b]], bufa.at[slot], asem).wait()
        pltpu.make_async_copy(
            aad_hbm.at[didx.at[b]], bufd.at[slot], dsem).wait()

        @pl.when(b >= 2)
        def _():
            pltpu.make_async_copy(exb.at[slot], ex_hbm.at[w, b - 2], osem).wait()

        @pl.when(b + 1 < NBLK)
        def _():
            pltpu.async_copy(aas_hbm.at[sidx.at[b + 1]], bufa.at[1 - slot], asem)
            pltpu.async_copy(aad_hbm.at[didx.at[b + 1]], bufd.at[1 - slot], dsem)

        def edge(r, carry2):
            e = bufa[slot, r, pl.ds(0, 16)] + bufd[slot, r, pl.ds(0, 16)]
            e = jnp.where(e >= 0.0, e, 0.2 * e)
            exb[slot, r, pl.ds(0, 16)] = jnp.exp(e - cc)
            return carry2

        lax.fori_loop(0, EB, edge, carry)
        pltpu.async_copy(exb.at[slot], ex_hbm.at[w, b], osem)
        return carry

    lax.fori_loop(0, NBLK, blk, 0)
    pltpu.make_async_copy(exb.at[lax.rem(NBLK - 2, 2)],
                          ex_hbm.at[w, NBLK - 2], osem).wait()
    pltpu.make_async_copy(exb.at[lax.rem(NBLK - 1, 2)],
                          ex_hbm.at[w, NBLK - 1], osem).wait()


@functools.partial(
    pl.kernel,
    out_type=jax.ShapeDtypeStruct((NC, NPAD, 128), F32),
    mesh=_mesh(),
    scratch_types=[
        pltpu.VMEM((NBLK, EB), I32),
        pltpu.VMEM((EB, 16), F32),
        pltpu.VMEM((EB, 128), F32),
        pltpu.VMEM_SHARED((NPAD, 128), F32),
        pltpu.SemaphoreType.DMA,
    ],
)
def _gats_kernel(ex_hbm, dst_hbm, zeros_hbm, szero_hbm, out_hbm,
                 didx, exr, sprep, acc, ssem):
    c = lax.axis_index("c")
    s = lax.axis_index("s")
    w = s * NC + c
    pltpu.sync_copy(dst_hbm.at[w], didx)
    pltpu.sync_copy(zeros_hbm, acc.at[pl.ds(s * RT, RT)])
    pltpu.sync_copy(szero_hbm, sprep)
    plsc.subcore_barrier()

    def blk(b, carry):
        pltpu.sync_copy(ex_hbm.at[w, b], exr)

        @pl.when(b >= 1)
        def _():
            pltpu.make_async_copy(
                sprep, acc.at[didx.at[b - 1]], ssem).wait()

        def edge(r, carry2):
            sprep[r, pl.ds(0, 16)] = exr[r, pl.ds(0, 16)]
            return carry2

        lax.fori_loop(0, EB, edge, carry)
        pltpu.async_copy(sprep, acc.at[didx.at[b]], ssem, add=True)
        return carry

    lax.fori_loop(0, NBLK, blk, 0)
    pltpu.make_async_copy(sprep, acc.at[didx.at[NBLK - 1]], ssem).wait()
    plsc.subcore_barrier()
    pltpu.sync_copy(acc.at[pl.ds(s * RT, RT)], out_hbm.at[c, pl.ds(s * RT, RT)])


@functools.partial(
    pl.kernel,
    out_type=jax.ShapeDtypeStruct((HEADS, NC, NPAD, 128), F32),
    mesh=_mesh(),
    scratch_types=[
        pltpu.VMEM((RING, EB), I32),
        pltpu.VMEM((NBLK, EB), I32),
        pltpu.VMEM((RING, EB), F32),
        pltpu.VMEM((2, EB, 128), F32),
        pltpu.VMEM_SHARED((NPAD, 128), F32),
        pltpu.SemaphoreType.DMA,
        pltpu.SemaphoreType.DMA,
        pltpu.SemaphoreType.DMA,
        pltpu.SemaphoreType.DMA,
    ],
)
def _gat_kernel(xh_hbm, ex_hbm, src_hbm, dst_hbm, zeros_hbm, out_hbm,
                sring, didx, exr, rows, acc, isem, esem, gsem, ssem):
    c = lax.axis_index("c")
    s = lax.axis_index("s")
    w = s * NC + c
    pltpu.sync_copy(dst_hbm.at[w], didx)

    for h in range(HEADS):
        src_tab = xh_hbm.at[h]
        pltpu.sync_copy(zeros_hbm, acc.at[pl.ds(s * RT, RT)])
        for k in range(RING):
            pltpu.async_copy(src_hbm.at[w, k], sring.at[k], isem)
            pltpu.async_copy(ex_hbm.at[w, k, h], exr.at[k], esem)
        plsc.subcore_barrier()

        pltpu.make_async_copy(src_hbm.at[w, 0], sring.at[0], isem).wait()
        pltpu.async_copy(src_tab.at[sring.at[0]], rows.at[0], gsem)

        def blk(b, carry):
            slot = lax.rem(b, 2)
            ib = lax.rem(b, RING)
            ibn = lax.rem(b + 1, RING)
            pltpu.make_async_copy(
                src_tab.at[sring.at[ib]], rows.at[slot], gsem).wait()

            @pl.when(b >= 1)
            def _():
                pltpu.make_async_copy(
                    rows.at[1 - slot], acc.at[didx.at[b - 1]], ssem).wait()

            @pl.when(b + 1 < NBLK)
            def _():
                pltpu.make_async_copy(
                    src_hbm.at[w, b + 1], sring.at[ibn], isem).wait()
                pltpu.async_copy(
                    src_tab.at[sring.at[ibn]], rows.at[1 - slot], gsem)

            pltpu.make_async_copy(
                ex_hbm.at[w, b, h], exr.at[ib], esem).wait()

            def grp(g, carry2):
                exvec = exr[ib, pl.ds(g * 16, 16)]
                for j in range(16):
                    exs = exvec[j]
                    r = g * 16 + j
                    for f in range(8):
                        v = rows[slot, r, pl.ds(f * 16, 16)]
                        rows[slot, r, pl.ds(f * 16, 16)] = v * exs
                return carry2

            lax.fori_loop(0, EB // 16, grp, 0)
            pltpu.async_copy(rows.at[slot], acc.at[didx.at[b]], ssem, add=True)

            @pl.when(b + RING < NBLK)
            def _():
                pltpu.async_copy(src_hbm.at[w, b + RING], sring.at[ib], isem)
                pltpu.async_copy(ex_hbm.at[w, b + RING, h], exr.at[ib], esem)

            return carry

        lax.fori_loop(0, NBLK, blk, 0)
        pltpu.make_async_copy(rows.at[lax.rem(NBLK - 1, 2)],
                              acc.at[didx.at[NBLK - 1]], ssem).wait()
        plsc.subcore_barrier()
        pltpu.sync_copy(acc.at[pl.ds(s * RT, RT)],
                        out_hbm.at[h, c, pl.ds(s * RT, RT)])
        plsc.subcore_barrier()


@functools.partial(
    pl.kernel,
    out_type=(jax.ShapeDtypeStruct((NC, GPAD, 128), F32),
              jax.ShapeDtypeStruct((NC, GPAD, 128), F32)),
    mesh=_mesh(),
    scratch_types=[
        pltpu.VMEM((5, 64), I32),
        pltpu.VMEM((64, 128), F32),
        pltpu.VMEM((64, 128), F32),
        pltpu.VMEM_SHARED((GPAD, 128), F32),
        pltpu.VMEM_SHARED((GPAD, 128), F32),
    ],
)
def _pool_kernel(x_hbm, bid_hbm, zeros_hbm, ones_hbm, outf_hbm, outc_hbm,
                 bidx, rowsv, onesv, accf, accc):
    c = lax.axis_index("c")
    s = lax.axis_index("s")
    w = s * NC + c
    pltpu.sync_copy(bid_hbm.at[w], bidx)
    pltpu.sync_copy(ones_hbm, onesv)
    pltpu.sync_copy(zeros_hbm, accf.at[pl.ds(s * GRT, GRT)])
    pltpu.sync_copy(zeros_hbm, accc.at[pl.ds(s * GRT, GRT)])
    plsc.subcore_barrier()

    for j in range(5):
        pltpu.sync_copy(x_hbm.at[pl.ds(w * PRT + j * 64, 64)], rowsv)
        pltpu.sync_copy(rowsv, accf.at[bidx.at[j]], add=True)
        pltpu.sync_copy(onesv, accc.at[bidx.at[j]], add=True)

    plsc.subcore_barrier()
    pltpu.sync_copy(accf.at[pl.ds(s * GRT, GRT)],
                    outf_hbm.at[c, pl.ds(s * GRT, GRT)])
    pltpu.sync_copy(accc.at[pl.ds(s * GRT, GRT)],
                    outc_hbm.at[c, pl.ds(s * GRT, GRT)])


_RB = 400
_NRB = N // _RB


def _tc_first(deg2_ref, x_ref, w0t_ref, dinv_ref, xt_ref):
    deg = deg2_ref[0] + deg2_ref[1]
    dinv = lax.rsqrt(jnp.maximum(deg, 1e-12))
    dinv_ref[...] = dinv
    xt_ref[...] = jnp.dot(x_ref[...], w0t_ref[...],
                          preferred_element_type=F32) * dinv


def _tc_mid(acc2_ref, dinv_ref, xs_ref, cst_ref, wnt_ref, xsn_ref, xtn_ref,
            *, residual):
    dinv = dinv_ref[...]
    h = (acc2_ref[0] + acc2_ref[1]) * dinv + cst_ref[0:1, :]
    h = h * cst_ref[1:2, :] + cst_ref[2:3, :]
    h = jnp.maximum(h, 0.0)
    if residual:
        h = xs_ref[...] + h
    xsn_ref[...] = h
    xtn_ref[...] = jnp.dot(h, wnt_ref[...], preferred_element_type=F32) * dinv


def _tc_last(acc2_ref, dinv_ref, xs_ref, cst_ref, wgt_ref, bmat_ref,
             xsn_ref, aa_ref, cm_ref):
    i = pl.program_id(0)
    h = (acc2_ref[0] + acc2_ref[1]) * dinv_ref[...] + cst_ref[0:1, :]
    h = h * cst_ref[1:2, :] + cst_ref[2:3, :]
    h = jnp.maximum(h, 0.0)
    h = xs_ref[...] + h
    xsn_ref[...] = h
    xh = jnp.dot(h, wgt_ref[...], preferred_element_type=F32)
    aa = jnp.dot(xh, bmat_ref[...], preferred_element_type=F32)
    aa_ref[...] = aa
    bmax = jnp.max(aa, axis=0, keepdims=True)

    @pl.when(i == 0)
    def _():
        cm_ref[...] = jnp.full((8, 128), -1e30, F32)

    cm_ref[...] = jnp.maximum(cm_ref[...], bmax)


def _tc_xh(xs_ref, wgt_ref, xh_ref):
    xh_ref[0] = jnp.dot(xs_ref[...], wgt_ref[0], preferred_element_type=F32)


def _tc_gat_combine(gf_ref, gs_ref, xs_ref, bg_ref, out_ref):
    accum = jnp.zeros((_RB, 128), F32)
    for h in range(HEADS):
        sden = gs_ref[0, :, h:h + 1] + gs_ref[1, :, h:h + 1]
        accum = accum + (gf_ref[h, 0] + gf_ref[h, 1]) / (sden + 1e-16)
    out_ref[...] = xs_ref[...] + accum * 0.25 + bg_ref[0:1, :]


def _tc_mlp(pf_ref, pc_ref, w1t_ref, b1_ref, w2t_ref, b2_ref, w3t_ref, b3_ref,
            out_ref):
    p = pf_ref[0] + pf_ref[1]
    cnt = jnp.maximum(pc_ref[0, :, 0:1] + pc_ref[1, :, 0:1], 1.0)
    pooled = p / cnt
    h1 = jnp.maximum(jnp.dot(pooled, w1t_ref[...],
                             preferred_element_type=F32) + b1_ref[0:1, :], 0.0)
    h2 = jnp.maximum(jnp.dot(h1, w2t_ref[...],
                             preferred_element_type=F32) + b2_ref[0:1, :], 0.0)
    out_ref[...] = jnp.dot(h2, w3t_ref[...],
                           preferred_element_type=F32) + b3_ref[0:1, :]


def _row_spec(lead=()):
    nlead = len(lead)
    return pl.BlockSpec(lead + (_RB, 128),
                        lambda i: (0,) * nlead + (i, 0))


def _full_spec(shape):
    ndim = len(shape)
    return pl.BlockSpec(shape, lambda i: (0,) * ndim)


def kernel(x, edge_index, batch, W0, b0, W1, b1, W2, b2, bn_gamma, bn_beta,
           Wg, att_src, att_dst, bg, Wc1, bc1, Wc2, bc2, Wc3, bc3):
    f32 = F32
    bn_scale = (bn_gamma * (1.0 / jnp.sqrt(jnp.float32(1.0 + 1e-5)))).astype(f32)

    loop = jnp.arange(N, dtype=jnp.int32)
    src2 = jnp.concatenate([edge_index[0], loop])
    dst2 = jnp.concatenate([edge_index[1], loop])
    pad = E2P - (E + N)
    srcp = jnp.concatenate([src2, jnp.zeros((pad,), jnp.int32)])
    dstp = jnp.concatenate([dst2, jnp.full((pad,), N, jnp.int32)])
    srcB = srcp.reshape(NW, NBLK, EB)
    dstB = dstp.reshape(NW, NBLK, EB)

    ones_rows = jnp.ones((EB, 128), f32)
    z128 = jnp.zeros((RT, 128), f32)
    zeb = jnp.zeros((EB, 128), f32)
    z256 = jnp.zeros((GRT, 128), f32)
    z64p = jnp.ones((64, 128), f32)

    degp = _deg_kernel(dstB, ones_rows, z128)
    deg2 = degp[:, :N, :]

    cst = [jnp.concatenate(
        [b[None, :], bn_scale[i][None, :], bn_beta[i][None, :],
         jnp.zeros((5, H), f32)], axis=0)
        for i, b in enumerate((b0, b1, b2))]

    dinvF, xt0 = pl.pallas_call(
        _tc_first,
        grid=(_NRB,),
        in_specs=[_row_spec((NC,)), _row_spec(), _full_spec((128, 128))],
        out_specs=[_row_spec(), _row_spec()],
        out_shape=[jax.ShapeDtypeStruct((N, 128), f32)] * 2,
    )(deg2, x, W0.T)

    acc1 = _gcn_kernel(xt0, srcB, dstB, z128)[:, :N, :]
    xs1, xt1 = pl.pallas_call(
        functools.partial(_tc_mid, residual=False),
        grid=(_NRB,),
        in_specs=[_row_spec((NC,)), _row_spec(), _row_spec(),
                  _full_spec((8, 128)), _full_spec((128, 128))],
        out_specs=[_row_spec(), _row_spec()],
        out_shape=[jax.ShapeDtypeStruct((N, 128), f32)] * 2,
    )(acc1, dinvF, x, cst[0], W1.T)

    acc2 = _gcn_kernel(xt1, srcB, dstB, z128)[:, :N, :]
    xs2, xt2 = pl.pallas_call(
        functools.partial(_tc_mid, residual=True),
        grid=(_NRB,),
        in_specs=[_row_spec((NC,)), _row_spec(), _row_spec(),
                  _full_spec((8, 128)), _full_spec((128, 128))],
        out_specs=[_row_spec(), _row_spec()],
        out_shape=[jax.ShapeDtypeStruct((N, 128), f32)] * 2,
    )(acc2, dinvF, xs1, cst[1], W2.T)

    acc3 = _gcn_kernel(xt2, srcB, dstB, z128)[:, :N, :]

    bmat = jnp.zeros((HEADS * H, 128), f32)
    for hh in range(HEADS):
        bmat = bmat.at[hh * H:(hh + 1) * H, hh].set(att_src[hh])
        bmat = bmat.at[hh * H:(hh + 1) * H, 4 + hh].set(att_dst[hh])

    xs3, aa, cm = pl.pallas_call(
        _tc_last,
        grid=(_NRB,),
        in_specs=[_row_spec((NC,)), _row_spec(), _row_spec(),
                  _full_spec((8, 128)), _full_spec((128, HEADS * H)),
                  _full_spec((HEADS * H, 128))],
        out_specs=[_row_spec(), _row_spec(),
                   pl.BlockSpec((8, 128), lambda i: (0, 0))],
        out_shape=[jax.ShapeDtypeStruct((N, 128), f32),
                   jax.ShapeDtypeStruct((N, 128), f32),
                   jax.ShapeDtypeStruct((8, 128), f32)],
    )(acc3, dinvF, xs2, cst[2], Wg.T, bmat)

    xh = pl.pallas_call(
        _tc_xh,
        grid=(HEADS, _NRB),
        in_specs=[pl.BlockSpec((_RB, 128), lambda h, i: (i, 0)),
                  pl.BlockSpec((1, 128, 128), lambda h, i: (h, 0, 0))],
        out_specs=pl.BlockSpec((1, _RB, 128), lambda h, i: (h, i, 0)),
        out_shape=jax.ShapeDtypeStruct((HEADS, N, 128), f32),
    )(xs3, Wg.T.reshape(128, HEADS, H).transpose(1, 0, 2))

    ms = cm[0, :HEADS]
    md = cm[0, HEADS:2 * HEADS]
    csum = ms + md
    cbound = jnp.max(jnp.where(csum >= 0.0, csum, 0.2 * csum))
    c16 = jnp.broadcast_to(cbound, (16,)).astype(f32)

    aad = jnp.pad(aa[:, 4:8], ((0, 0), (0, 124)))
    exq4 = _ex_kernel(aa, aad, c16, srcB, dstB)
    exT = jnp.swapaxes(exq4, 2, 3)
    gaccs = _gats_kernel(exq4, dstB, z128, zeb)[:, :N, :]
    gaccf = _gat_kernel(xh, exT, srcB, dstB, z128)[:, :, :N, :]

    bgc = jnp.concatenate([bg[None, :], jnp.zeros((7, H), f32)], axis=0)
    xs4 = pl.pallas_call(
        _tc_gat_combine,
        grid=(_NRB,),
        in_specs=[pl.BlockSpec((HEADS, NC, _RB, 128),
                               lambda i: (0, 0, i, 0)),
                  _row_spec((NC,)), _row_spec(), _full_spec((8, 128))],
        out_specs=_row_spec(),
        out_shape=jax.ShapeDtypeStruct((N, 128), f32),
    )(gaccf, gaccs, xs3, bgc)

    xs4p = jnp.concatenate([xs4, jnp.zeros((PP - N, 128), f32)], axis=0)
    bidp = jnp.concatenate(
        [batch.astype(jnp.int32), jnp.full((PP - N,), G, jnp.int32)]
    ).reshape(NW, 5, 64)
    poolf, poolc = _pool_kernel(xs4p, bidp, z256, z64p)
    poolf = poolf[:, :G, :]
    poolc = poolc[:, :G, :]

    out = pl.pallas_call(
        _tc_mlp,
        grid=(1,),
        in_specs=[_full_spec((NC, G, 128)), _full_spec((NC, G, 128)),
                  _full_spec((128, 64)),
                  _full_spec((1, 64)), _full_spec((64, 32)),
                  _full_spec((1, 32)), _full_spec((32, 1)),
                  _full_spec((1, 1))],
        out_specs=_full_spec((G, 1)),
        out_shape=jax.ShapeDtypeStruct((G, 1), f32),
    )(poolf, poolc, Wc1.T, bc1[None, :], Wc2.T, bc2[None, :], Wc3.T,
      bc3[None, :])

    return out


# speedup vs baseline: 19.5496x; 1.2211x over previous
import functools

import jax
import jax.numpy as jnp
from jax import lax
from jax.experimental import pallas as pl
from jax.experimental.pallas import tpu as pltpu
from jax.experimental.pallas import tpu_sc as plsc

N = 10000
E = 320000
D = 128
H = 128
HEADS = 4
G = 512

NC = 2
NS = 16
NW = NC * NS
EB = 128
NBLK = 81
EPW = EB * NBLK
E2P = NW * EPW
NPAD = 10112
RT = NPAD // NS
RING = 8
PP = 10240
PRT = PP // NW
GPAD = 640
GRT = GPAD // NS
PW = 256

F32 = jnp.float32
I32 = jnp.int32


def _mesh():
    return plsc.VectorSubcoreMesh(
        core_axis_name="c", subcore_axis_name="s",
        num_cores=NC, num_subcores=NS)


@functools.partial(
    pl.kernel,
    out_type=jax.ShapeDtypeStruct((NC, NPAD, 128), F32),
    mesh=_mesh(),
    scratch_types=[
        pltpu.VMEM((EB, 128), F32),
        pltpu.VMEM((NBLK, EB), I32),
        pltpu.VMEM_SHARED((NPAD, 128), F32),
        pltpu.SemaphoreType.DMA,
    ],
)
def _deg_kernel(dst_hbm, ones_hbm, zeros_hbm, out_hbm, onesv, didx, acc, ssem):
    c = lax.axis_index("c")
    s = lax.axis_index("s")
    w = s * NC + c
    pltpu.sync_copy(ones_hbm, onesv)
    pltpu.sync_copy(dst_hbm.at[w], didx)
    pltpu.sync_copy(zeros_hbm, acc.at[pl.ds(s * RT, RT)])
    plsc.subcore_barrier()

    def blk(b, carry):
        pltpu.async_copy(onesv, acc.at[didx.at[b]], ssem, add=True)
        return carry

    lax.fori_loop(0, NBLK, blk, 0)

    def drain(b, carry):
        pltpu.make_async_copy(onesv, acc.at[didx.at[b]], ssem).wait()
        return carry

    lax.fori_loop(0, NBLK, drain, 0)
    plsc.subcore_barrier()
    pltpu.sync_copy(acc.at[pl.ds(s * RT, RT)], out_hbm.at[c, pl.ds(s * RT, RT)])


@functools.partial(
    pl.kernel,
    out_type=jax.ShapeDtypeStruct((NC, NPAD, 128), F32),
    mesh=_mesh(),
    scratch_types=[
        pltpu.VMEM((RING, EB), I32),
        pltpu.VMEM((NBLK, EB), I32),
        pltpu.VMEM((2, EB, 128), F32),
        pltpu.VMEM_SHARED((NPAD, 128), F32),
        pltpu.SemaphoreType.DMA,
        pltpu.SemaphoreType.DMA,
        pltpu.SemaphoreType.DMA,
    ],
)
def _gcn_kernel(xt_hbm, src_hbm, dst_hbm, zeros_hbm, out_hbm,
                sring, didx, rows, acc, isem, gsem, ssem):
    c = lax.axis_index("c")
    s = lax.axis_index("s")
    w = s * NC + c
    pltpu.sync_copy(dst_hbm.at[w], didx)
    pltpu.sync_copy(zeros_hbm, acc.at[pl.ds(s * RT, RT)])
    for k in range(RING):
        pltpu.async_copy(src_hbm.at[w, k], sring.at[k], isem)
    plsc.subcore_barrier()

    pltpu.make_async_copy(src_hbm.at[w, 0], sring.at[0], isem).wait()
    pltpu.async_copy(xt_hbm.at[sring.at[0]], rows.at[0], gsem)

    def blk(b, carry):
        slot = lax.rem(b, 2)
        ib = lax.rem(b, RING)
        ibn = lax.rem(b + 1, RING)
        pltpu.make_async_copy(
            xt_hbm.at[sring.at[ib]], rows.at[slot], gsem).wait()

        @pl.when(b >= 1)
        def _():
            pltpu.make_async_copy(
                rows.at[1 - slot], acc.at[didx.at[b - 1]], ssem).wait()

        @pl.when(b + 1 < NBLK)
        def _():
            pltpu.make_async_copy(
                src_hbm.at[w, b + 1], sring.at[ibn], isem).wait()
            pltpu.async_copy(xt_hbm.at[sring.at[ibn]], rows.at[1 - slot], gsem)

        pltpu.async_copy(rows.at[slot], acc.at[didx.at[b]], ssem, add=True)

        @pl.when(b + RING < NBLK)
        def _():
            pltpu.async_copy(src_hbm.at[w, b + RING], sring.at[ib], isem)

        return carry

    lax.fori_loop(0, NBLK, blk, 0)
    pltpu.make_async_copy(rows.at[lax.rem(NBLK - 1, 2)],
                          acc.at[didx.at[NBLK - 1]], ssem).wait()
    plsc.subcore_barrier()
    pltpu.sync_copy(acc.at[pl.ds(s * RT, RT)], out_hbm.at[c, pl.ds(s * RT, RT)])


@functools.partial(
    pl.kernel,
    out_type=jax.ShapeDtypeStruct((NW, NBLK, EB, 16), F32),
    mesh=_mesh(),
    scratch_types=[
        pltpu.VMEM((16,), F32),
        pltpu.VMEM((NBLK, EB), I32),
        pltpu.VMEM((NBLK, EB), I32),
        pltpu.VMEM((2, EB, 128), F32),
        pltpu.VMEM((2, EB, 128), F32),
        pltpu.VMEM((2, EB, 16), F32),
        pltpu.SemaphoreType.DMA,
        pltpu.SemaphoreType.DMA,
        pltpu.SemaphoreType.DMA,
    ],
)
def _ex_kernel(aas_hbm, aad_hbm, c_hbm, src_hbm, dst_hbm, ex_hbm,
               cv, sidx, didx, bufa, bufd, exb, asem, dsem, osem):
    c = lax.axis_index("c")
    s = lax.axis_index("s")
    w = s * NC + c
    pltpu.sync_copy(c_hbm, cv)
    pltpu.sync_copy(src_hbm.at[w], sidx)
    pltpu.sync_copy(dst_hbm.at[w], didx)
    cc = cv[...]
    pltpu.async_copy(aas_hbm.at[sidx.at[0]], bufa.at[0], asem)
    pltpu.async_copy(aad_hbm.at[didx.at[0]], bufd.at[0], dsem)

    def blk(b, carry):
        slot = lax.rem(b, 2)
        pltpu.make_async_copy(
            aas_hbm.at[sidx.at[b]], bufa.at[slot], asem).wait()
        pltpu.make_async_copy(
            aad_hbm.at[didx.at[b]], bufd.at[slot], dsem).wait()

        @pl.when(b >= 2)
        def _():
            pltpu.make_async_copy(exb.at[slot], ex_hbm.at[w, b - 2], osem).wait()

        @pl.when(b + 1 < NBLK)
        def _():
            pltpu.async_copy(aas_hbm.at[sidx.at[b + 1]], bufa.at[1 - slot], asem)
            pltpu.async_copy(aad_hbm.at[didx.at[b + 1]], bufd.at[1 - slot], dsem)

        @plsc.parallel_loop(0, EB, unroll=4)
        def _(r):
            e = bufa[slot, r, pl.ds(0, 16)] + bufd[slot, r, pl.ds(0, 16)]
            e = jnp.where(e >= 0.0, e, 0.2 * e)
            exb[slot, r, pl.ds(0, 16)] = jnp.exp(e - cc)
        pltpu.async_copy(exb.at[slot], ex_hbm.at[w, b], osem)
        return carry

    lax.fori_loop(0, NBLK, blk, 0)
    pltpu.make_async_copy(exb.at[lax.rem(NBLK - 2, 2)],
                          ex_hbm.at[w, NBLK - 2], osem).wait()
    pltpu.make_async_copy(exb.at[lax.rem(NBLK - 1, 2)],
                          ex_hbm.at[w, NBLK - 1], osem).wait()


@functools.partial(
    pl.kernel,
    out_type=jax.ShapeDtypeStruct((NC, NPAD, 128), F32),
    mesh=_mesh(),
    scratch_types=[
        pltpu.VMEM((NBLK, EB), I32),
        pltpu.VMEM((EB, 16), F32),
        pltpu.VMEM((EB, 128), F32),
        pltpu.VMEM_SHARED((NPAD, 128), F32),
        pltpu.SemaphoreType.DMA,
    ],
)
def _gats_kernel(ex_hbm, dst_hbm, zeros_hbm, szero_hbm, out_hbm,
                 didx, exr, sprep, acc, ssem):
    c = lax.axis_index("c")
    s = lax.axis_index("s")
    w = s * NC + c
    pltpu.sync_copy(dst_hbm.at[w], didx)
    pltpu.sync_copy(zeros_hbm, acc.at[pl.ds(s * RT, RT)])
    pltpu.sync_copy(szero_hbm, sprep)
    plsc.subcore_barrier()

    def blk(b, carry):
        pltpu.sync_copy(ex_hbm.at[w, b], exr)

        @pl.when(b >= 1)
        def _():
            pltpu.make_async_copy(
                sprep, acc.at[didx.at[b - 1]], ssem).wait()

        @plsc.parallel_loop(0, EB, unroll=4)
        def _(r):
            sprep[r, pl.ds(0, 16)] = exr[r, pl.ds(0, 16)]
        pltpu.async_copy(sprep, acc.at[didx.at[b]], ssem, add=True)
        return carry

    lax.fori_loop(0, NBLK, blk, 0)
    pltpu.make_async_copy(sprep, acc.at[didx.at[NBLK - 1]], ssem).wait()
    plsc.subcore_barrier()
    pltpu.sync_copy(acc.at[pl.ds(s * RT, RT)], out_hbm.at[c, pl.ds(s * RT, RT)])


@functools.partial(
    pl.kernel,
    out_type=jax.ShapeDtypeStruct((HEADS, NC, NPAD, 128), F32),
    mesh=_mesh(),
    scratch_types=[
        pltpu.VMEM((RING, EB), I32),
        pltpu.VMEM((NBLK, EB), I32),
        pltpu.VMEM((RING, EB), F32),
        pltpu.VMEM((2, EB, 128), F32),
        pltpu.VMEM_SHARED((NPAD, 128), F32),
        pltpu.SemaphoreType.DMA,
        pltpu.SemaphoreType.DMA,
        pltpu.SemaphoreType.DMA,
        pltpu.SemaphoreType.DMA,
    ],
)
def _gat_kernel(xh_hbm, ex_hbm, src_hbm, dst_hbm, zeros_hbm, out_hbm,
                sring, didx, exr, rows, acc, isem, esem, gsem, ssem):
    c = lax.axis_index("c")
    s = lax.axis_index("s")
    w = s * NC + c
    pltpu.sync_copy(dst_hbm.at[w], didx)

    for h in range(HEADS):
        src_tab = xh_hbm.at[h]
        pltpu.sync_copy(zeros_hbm, acc.at[pl.ds(s * RT, RT)])
        for k in range(RING):
            pltpu.async_copy(src_hbm.at[w, k], sring.at[k], isem)
            pltpu.async_copy(ex_hbm.at[w, k, h], exr.at[k], esem)
        plsc.subcore_barrier()

        pltpu.make_async_copy(src_hbm.at[w, 0], sring.at[0], isem).wait()
        pltpu.async_copy(src_tab.at[sring.at[0]], rows.at[0], gsem)

        def blk(b, carry):
            slot = lax.rem(b, 2)
            ib = lax.rem(b, RING)
            ibn = lax.rem(b + 1, RING)
            pltpu.make_async_copy(
                src_tab.at[sring.at[ib]], rows.at[slot], gsem).wait()

            @pl.when(b >= 1)
            def _():
                pltpu.make_async_copy(
                    rows.at[1 - slot], acc.at[didx.at[b - 1]], ssem).wait()

            @pl.when(b + 1 < NBLK)
            def _():
                pltpu.make_async_copy(
                    src_hbm.at[w, b + 1], sring.at[ibn], isem).wait()
                pltpu.async_copy(
                    src_tab.at[sring.at[ibn]], rows.at[1 - slot], gsem)

            pltpu.make_async_copy(
                ex_hbm.at[w, b, h], exr.at[ib], esem).wait()

            @plsc.parallel_loop(0, EB // 16, unroll=2)
            def _(g):
                exvec = exr[ib, pl.ds(g * 16, 16)]
                for j in range(16):
                    exs = exvec[j]
                    r = g * 16 + j
                    for f in range(8):
                        v = rows[slot, r, pl.ds(f * 16, 16)]
                        rows[slot, r, pl.ds(f * 16, 16)] = v * exs
            pltpu.async_copy(rows.at[slot], acc.at[didx.at[b]], ssem, add=True)

            @pl.when(b + RING < NBLK)
            def _():
                pltpu.async_copy(src_hbm.at[w, b + RING], sring.at[ib], isem)
                pltpu.async_copy(ex_hbm.at[w, b + RING, h], exr.at[ib], esem)

            return carry

        lax.fori_loop(0, NBLK, blk, 0)
        pltpu.make_async_copy(rows.at[lax.rem(NBLK - 1, 2)],
                              acc.at[didx.at[NBLK - 1]], ssem).wait()
        plsc.subcore_barrier()
        pltpu.sync_copy(acc.at[pl.ds(s * RT, RT)],
                        out_hbm.at[h, c, pl.ds(s * RT, RT)])
        plsc.subcore_barrier()


@functools.partial(
    pl.kernel,
    out_type=(jax.ShapeDtypeStruct((NC, GPAD, 128), F32),
              jax.ShapeDtypeStruct((NC, GPAD, 128), F32)),
    mesh=_mesh(),
    scratch_types=[
        pltpu.VMEM((5, 64), I32),
        pltpu.VMEM((64, 128), F32),
        pltpu.VMEM((64, 128), F32),
        pltpu.VMEM_SHARED((GPAD, 128), F32),
        pltpu.VMEM_SHARED((GPAD, 128), F32),
    ],
)
def _pool_kernel(x_hbm, bid_hbm, zeros_hbm, ones_hbm, outf_hbm, outc_hbm,
                 bidx, rowsv, onesv, accf, accc):
    c = lax.axis_index("c")
    s = lax.axis_index("s")
    w = s * NC + c
    pltpu.sync_copy(bid_hbm.at[w], bidx)
    pltpu.sync_copy(ones_hbm, onesv)
    pltpu.sync_copy(zeros_hbm, accf.at[pl.ds(s * GRT, GRT)])
    pltpu.sync_copy(zeros_hbm, accc.at[pl.ds(s * GRT, GRT)])
    plsc.subcore_barrier()

    for j in range(5):
        pltpu.sync_copy(x_hbm.at[pl.ds(w * PRT + j * 64, 64)], rowsv)
        pltpu.sync_copy(rowsv, accf.at[bidx.at[j]], add=True)
        pltpu.sync_copy(onesv, accc.at[bidx.at[j]], add=True)

    plsc.subcore_barrier()
    pltpu.sync_copy(accf.at[pl.ds(s * GRT, GRT)],
                    outf_hbm.at[c, pl.ds(s * GRT, GRT)])
    pltpu.sync_copy(accc.at[pl.ds(s * GRT, GRT)],
                    outc_hbm.at[c, pl.ds(s * GRT, GRT)])


_RB = 400
_NRB = N // _RB


def _tc_first(deg2_ref, x_ref, w0t_ref, dinv_ref, xt_ref):
    deg = deg2_ref[0] + deg2_ref[1]
    dinv = lax.rsqrt(jnp.maximum(deg, 1e-12))
    dinv_ref[...] = dinv
    xt_ref[...] = jnp.dot(x_ref[...], w0t_ref[...],
                          preferred_element_type=F32) * dinv


def _tc_mid(acc2_ref, dinv_ref, xs_ref, cst_ref, wnt_ref, xsn_ref, xtn_ref,
            *, residual):
    dinv = dinv_ref[...]
    h = (acc2_ref[0] + acc2_ref[1]) * dinv + cst_ref[0:1, :]
    h = h * cst_ref[1:2, :] + cst_ref[2:3, :]
    h = jnp.maximum(h, 0.0)
    if residual:
        h = xs_ref[...] + h
    xsn_ref[...] = h
    xtn_ref[...] = jnp.dot(h, wnt_ref[...], preferred_element_type=F32) * dinv


def _tc_last(acc2_ref, dinv_ref, xs_ref, cst_ref, wgt_ref, bmat_ref,
             xsn_ref, aa_ref, cm_ref):
    i = pl.program_id(0)
    h = (acc2_ref[0] + acc2_ref[1]) * dinv_ref[...] + cst_ref[0:1, :]
    h = h * cst_ref[1:2, :] + cst_ref[2:3, :]
    h = jnp.maximum(h, 0.0)
    h = xs_ref[...] + h
    xsn_ref[...] = h
    xh = jnp.dot(h, wgt_ref[...], preferred_element_type=F32)
    aa = jnp.dot(xh, bmat_ref[...], preferred_element_type=F32)
    aa_ref[...] = aa
    bmax = jnp.max(aa, axis=0, keepdims=True)

    @pl.when(i == 0)
    def _():
        cm_ref[...] = jnp.full((8, 128), -1e30, F32)

    cm_ref[...] = jnp.maximum(cm_ref[...], bmax)


def _tc_xh(xs_ref, wgt_ref, xh_ref):
    xh_ref[0] = jnp.dot(xs_ref[...], wgt_ref[0], preferred_element_type=F32)


def _tc_gat_combine(gf_ref, gs_ref, xs_ref, bg_ref, out_ref):
    accum = jnp.zeros((_RB, 128), F32)
    for h in range(HEADS):
        sden = gs_ref[0, :, h:h + 1] + gs_ref[1, :, h:h + 1]
        accum = accum + (gf_ref[h, 0] + gf_ref[h, 1]) / (sden + 1e-16)
    out_ref[...] = xs_ref[...] + accum * 0.25 + bg_ref[0:1, :]


def _tc_mlp(pf_ref, pc_ref, w1t_ref, b1_ref, w2t_ref, b2_ref, w3t_ref, b3_ref,
            out_ref):
    p = pf_ref[0] + pf_ref[1]
    cnt = jnp.maximum(pc_ref[0, :, 0:1] + pc_ref[1, :, 0:1], 1.0)
    pooled = p / cnt
    h1 = jnp.maximum(jnp.dot(pooled, w1t_ref[...],
                             preferred_element_type=F32) + b1_ref[0:1, :], 0.0)
    h2 = jnp.maximum(jnp.dot(h1, w2t_ref[...],
                             preferred_element_type=F32) + b2_ref[0:1, :], 0.0)
    out_ref[...] = jnp.dot(h2, w3t_ref[...],
                           preferred_element_type=F32) + b3_ref[0:1, :]


def _row_spec(lead=()):
    nlead = len(lead)
    return pl.BlockSpec(lead + (_RB, 128),
                        lambda i: (0,) * nlead + (i, 0))


def _full_spec(shape):
    ndim = len(shape)
    return pl.BlockSpec(shape, lambda i: (0,) * ndim)


def kernel(x, edge_index, batch, W0, b0, W1, b1, W2, b2, bn_gamma, bn_beta,
           Wg, att_src, att_dst, bg, Wc1, bc1, Wc2, bc2, Wc3, bc3):
    f32 = F32
    bn_scale = (bn_gamma * (1.0 / jnp.sqrt(jnp.float32(1.0 + 1e-5)))).astype(f32)

    loop = jnp.arange(N, dtype=jnp.int32)
    src2 = jnp.concatenate([edge_index[0], loop])
    dst2 = jnp.concatenate([edge_index[1], loop])
    pad = E2P - (E + N)
    srcp = jnp.concatenate([src2, jnp.zeros((pad,), jnp.int32)])
    dstp = jnp.concatenate([dst2, jnp.full((pad,), N, jnp.int32)])
    srcB = srcp.reshape(NW, NBLK, EB)
    dstB = dstp.reshape(NW, NBLK, EB)

    ones_rows = jnp.ones((EB, 128), f32)
    z128 = jnp.zeros((RT, 128), f32)
    zeb = jnp.zeros((EB, 128), f32)
    z256 = jnp.zeros((GRT, 128), f32)
    z64p = jnp.ones((64, 128), f32)

    degp = _deg_kernel(dstB, ones_rows, z128)
    deg2 = degp[:, :N, :]

    cst = [jnp.concatenate(
        [b[None, :], bn_scale[i][None, :], bn_beta[i][None, :],
         jnp.zeros((5, H), f32)], axis=0)
        for i, b in enumerate((b0, b1, b2))]

    dinvF, xt0 = pl.pallas_call(
        _tc_first,
        grid=(_NRB,),
        in_specs=[_row_spec((NC,)), _row_spec(), _full_spec((128, 128))],
        out_specs=[_row_spec(), _row_spec()],
        out_shape=[jax.ShapeDtypeStruct((N, 128), f32)] * 2,
    )(deg2, x, W0.T)

    acc1 = _gcn_kernel(xt0, srcB, dstB, z128)[:, :N, :]
    xs1, xt1 = pl.pallas_call(
        functools.partial(_tc_mid, residual=False),
        grid=(_NRB,),
        in_specs=[_row_spec((NC,)), _row_spec(), _row_spec(),
                  _full_spec((8, 128)), _full_spec((128, 128))],
        out_specs=[_row_spec(), _row_spec()],
        out_shape=[jax.ShapeDtypeStruct((N, 128), f32)] * 2,
    )(acc1, dinvF, x, cst[0], W1.T)

    acc2 = _gcn_kernel(xt1, srcB, dstB, z128)[:, :N, :]
    xs2, xt2 = pl.pallas_call(
        functools.partial(_tc_mid, residual=True),
        grid=(_NRB,),
        in_specs=[_row_spec((NC,)), _row_spec(), _row_spec(),
                  _full_spec((8, 128)), _full_spec((128, 128))],
        out_specs=[_row_spec(), _row_spec()],
        out_shape=[jax.ShapeDtypeStruct((N, 128), f32)] * 2,
    )(acc2, dinvF, xs1, cst[1], W2.T)

    acc3 = _gcn_kernel(xt2, srcB, dstB, z128)[:, :N, :]

    bmat = jnp.zeros((HEADS * H, 128), f32)
    for hh in range(HEADS):
        bmat = bmat.at[hh * H:(hh + 1) * H, hh].set(att_src[hh])
        bmat = bmat.at[hh * H:(hh + 1) * H, 4 + hh].set(att_dst[hh])

    xs3, aa, cm = pl.pallas_call(
        _tc_last,
        grid=(_NRB,),
        in_specs=[_row_spec((NC,)), _row_spec(), _row_spec(),
                  _full_spec((8, 128)), _full_spec((128, HEADS * H)),
                  _full_spec((HEADS * H, 128))],
        out_specs=[_row_spec(), _row_spec(),
                   pl.BlockSpec((8, 128), lambda i: (0, 0))],
        out_shape=[jax.ShapeDtypeStruct((N, 128), f32),
                   jax.ShapeDtypeStruct((N, 128), f32),
                   jax.ShapeDtypeStruct((8, 128), f32)],
    )(acc3, dinvF, xs2, cst[2], Wg.T, bmat)

    xh = pl.pallas_call(
        _tc_xh,
        grid=(HEADS, _NRB),
        in_specs=[pl.BlockSpec((_RB, 128), lambda h, i: (i, 0)),
                  pl.BlockSpec((1, 128, 128), lambda h, i: (h, 0, 0))],
        out_specs=pl.BlockSpec((1, _RB, 128), lambda h, i: (h, i, 0)),
        out_shape=jax.ShapeDtypeStruct((HEADS, N, 128), f32),
    )(xs3, Wg.T.reshape(128, HEADS, H).transpose(1, 0, 2))

    ms = cm[0, :HEADS]
    md = cm[0, HEADS:2 * HEADS]
    csum = ms + md
    cbound = jnp.max(jnp.where(csum >= 0.0, csum, 0.2 * csum))
    c16 = jnp.broadcast_to(cbound, (16,)).astype(f32)

    aad = jnp.pad(aa[:, 4:8], ((0, 0), (0, 124)))
    exq4 = _ex_kernel(aa, aad, c16, srcB, dstB)
    exT = jnp.swapaxes(exq4, 2, 3)
    gaccs = _gats_kernel(exq4, dstB, z128, zeb)[:, :N, :]
    gaccf = _gat_kernel(xh, exT, srcB, dstB, z128)[:, :, :N, :]

    bgc = jnp.concatenate([bg[None, :], jnp.zeros((7, H), f32)], axis=0)
    xs4 = pl.pallas_call(
        _tc_gat_combine,
        grid=(_NRB,),
        in_specs=[pl.BlockSpec((HEADS, NC, _RB, 128),
                               lambda i: (0, 0, i, 0)),
                  _row_spec((NC,)), _row_spec(), _full_spec((8, 128))],
        out_specs=_row_spec(),
        out_shape=jax.ShapeDtypeStruct((N, 128), f32),
    )(gaccf, gaccs, xs3, bgc)

    xs4p = jnp.concatenate([xs4, jnp.zeros((PP - N, 128), f32)], axis=0)
    bidp = jnp.concatenate(
        [batch.astype(jnp.int32), jnp.full((PP - N,), G, jnp.int32)]
    ).reshape(NW, 5, 64)
    poolf, poolc = _pool_kernel(xs4p, bidp, z256, z64p)
    poolf = poolf[:, :G, :]
    poolc = poolc[:, :G, :]

    out = pl.pallas_call(
        _tc_mlp,
        grid=(1,),
        in_specs=[_full_spec((NC, G, 128)), _full_spec((NC, G, 128)),
                  _full_spec((128, 64)),
                  _full_spec((1, 64)), _full_spec((64, 32)),
                  _full_spec((1, 32)), _full_spec((32, 1)),
                  _full_spec((1, 1))],
        out_specs=_full_spec((G, 1)),
        out_shape=jax.ShapeDtypeStruct((G, 1), f32),
    )(poolf, poolc, Wc1.T, bc1[None, :], Wc2.T, bc2[None, :], Wc3.T,
      bc3[None, :])

    return out
